# baseline (speedup 1.0000x reference)
import jax
import jax.numpy as jnp
from jax import lax
from jax.experimental import pallas as pl
from jax.experimental.pallas import tpu as pltpu
from jax.experimental.pallas import tpu_sc as plsc

N = 10000
E = 320000
D_IN = 128
HEADS = 8
C = 8
PER_HOP = 64

NC = 2
NS = 16
NW = NC * NS
BLK = 64
SBLK = 512
EW = 10240
EPAD = EW * NW
NP = 10112
RPT = NP // NS
TCHUNK = 512
TCAP = 10544

OW = 128
EECOL = 112

_ROWS_TC = 1000


def _tc_a_body(x_ref, w1_ref, ap1_ref, adp1_ref, w2_ref, ap2_ref, adp2_ref,
               t1_ref, d1_ref, t2_ref, d2_ref):
    x = x_ref[...]
    for w_ref, ap_ref, adp_ref, t_ref, d_ref in (
            (w1_ref, ap1_ref, adp1_ref, t1_ref, d1_ref),
            (w2_ref, ap2_ref, adp2_ref, t2_ref, d2_ref)):
        h = jnp.dot(x, w_ref[...], preferred_element_type=jnp.float32)
        ab = jnp.dot(h, ap_ref[...], preferred_element_type=jnp.float32)
        t_ref[...] = jnp.concatenate([h, ab], axis=-1)
        d_ref[...] = jnp.dot(h, adp_ref[...], preferred_element_type=jnp.float32)


def _tc_a(x, W1, AP1, ADP1, W2, AP2, ADP2):
    grid = (N // _ROWS_TC,)
    rs = lambda w: pl.BlockSpec((_ROWS_TC, w), lambda i: (i, 0))
    full = lambda shape: pl.BlockSpec(shape, lambda i: (0, 0))
    return pl.pallas_call(
        _tc_a_body,
        grid=grid,
        in_specs=[rs(D_IN), full((D_IN, PER_HOP)), full((PER_HOP, PER_HOP)),
                  full((PER_HOP, 16)), full((D_IN, PER_HOP)),
                  full((PER_HOP, PER_HOP)), full((PER_HOP, 16))],
        out_specs=[rs(128), rs(16), rs(128), rs(16)],
        out_shape=[
            jax.ShapeDtypeStruct((N, 128), jnp.float32),
            jax.ShapeDtypeStruct((N, 16), jnp.float32),
            jax.ShapeDtypeStruct((N, 128), jnp.float32),
            jax.ShapeDtypeStruct((N, 16), jnp.float32),
        ],
    )(x, W1, AP1, ADP1, W2, AP2, ADP2)


def _vgather(x, idx):
    return lax.gather(
        x, idx[:, None],
        lax.GatherDimensionNumbers(offset_dims=(), collapsed_slice_dims=(0,),
                                   start_index_map=(0,)),
        (1,), mode=lax.GatherScatterMode.PROMISE_IN_BOUNDS)


def _sc_body(srcp, dstp, tab1, tab2, zh,
             p1out, p2out, acc,
             tpack_v, tsrc_v, tdst_v,
             gsrcb_v, gdstb_v, idxb_v,
             hrows_a, hrows_b, adst_a, adst_b,
             semg_a, semg_b, sems_a, sems_b):
    cc = lax.axis_index("c")
    ss = lax.axis_index("s")
    wid = cc * NS + ss
    ebase = wid * EW
    rb = ss * RPT
    iota = lax.iota(jnp.int32, 16)
    zv = iota * 0

    def zero_acc():
        pltpu.sync_copy(zh, acc.at[pl.ds(rb, RPT)])

    def zloop(i, _):
        tpack_v[pl.ds(i * 16, 16)] = zv
        return 0

    lax.fori_loop(0, TCAP // 16, zloop, 0, unroll=4)

    def t_chunk(ci, cnt):
        def live(cnt):
            base = pl.multiple_of(ci * TCHUNK, TCHUNK)
            pltpu.sync_copy(srcp.at[pl.ds(base, TCHUNK)], tsrc_v)
            pltpu.sync_copy(dstp.at[pl.ds(base, TCHUNK)], tdst_v)

            def vstep(k, cnt):
                sv = tsrc_v[pl.ds(k * 16, 16)]
                dv = tdst_v[pl.ds(k * 16, 16)]
                m = sv != dv
                cs = plsc.cumsum(m.astype(jnp.int32))
                pos = jnp.where(m, cnt + cs - 1, (TCAP - 16) + iota)
                plsc.store_scatter(tpack_v, [pos], sv + (dv << 16))
                return cnt + cs[15]

            return lax.fori_loop(0, TCHUNK // 16, vstep, cnt)

        return lax.cond(cnt < N, live, lambda cnt: cnt, cnt)

    lax.fori_loop(0, EPAD // TCHUNK, t_chunk, 0)

    hrows = (hrows_a, hrows_b)
    adst = (adst_a, adst_b)
    semg = (semg_a, semg_b)
    sems = (sems_a, sems_b)
    NBLKS = SBLK // BLK

    def run_hop(tab, pout, is_hop2):
        def superblock(sb, _):
            base = pl.multiple_of(ebase + sb * SBLK, SBLK)
            pltpu.sync_copy(srcp.at[pl.ds(base, SBLK)], tsrc_v)
            pltpu.sync_copy(dstp.at[pl.ds(base, SBLK)], tdst_v)

            def prep(k, _):
                sv = tsrc_v[pl.ds(k * 16, 16)]
                dv = tdst_v[pl.ds(k * 16, 16)]
                keep = sv != dv
                if is_hop2:
                    tp = plsc.load_gather(tpack_v, [dv])
                    s2 = tp & 0xFFFF
                    d2 = lax.shift_right_logical(tp, 16)
                    keep = jnp.logical_and(keep, s2 != d2)
                    gs, gd = s2, d2
                else:
                    gs, gd = sv, dv
                j = k // (BLK // 16)
                o = (k % (BLK // 16)) * 16
                gsrcb_v[j, pl.ds(o, 16)] = gs
                gdstb_v[j, pl.ds(o, 16)] = gd
                idxb_v[j, pl.ds(o, 16)] = jnp.where(keep, gd, N)
                return 0

            lax.fori_loop(0, SBLK // 16, prep, 0)

            def fire_g(j, p):
                pltpu.async_copy(tab.at[gsrcb_v.at[j]], hrows[p], semg[p])
                pltpu.async_copy(tab.at[gdstb_v.at[j]], adst[p], semg[p])

            def drain_g(j, p):
                pltpu.make_async_copy(tab.at[gsrcb_v.at[j]], hrows[p], semg[p]).wait()
                pltpu.make_async_copy(tab.at[gdstb_v.at[j]], adst[p], semg[p]).wait()

            def fire_s(j, p):
                pltpu.async_copy(hrows[p], acc.at[idxb_v.at[j]], sems[p], add=True)

            def drain_s(j, p):
                pltpu.make_async_copy(hrows[p], acc.at[idxb_v.at[j]], sems[p]).wait()

            def compute(p):
                hb, ab = hrows[p], adst[p]

                @plsc.parallel_loop(0, BLK, unroll=8)
                def _(e):
                    asrc = hb[e, pl.ds(64, 16)]
                    adv = ab[e, pl.ds(64, 16)]
                    adr = _vgather(adv, (iota + 8) & 15)
                    sc = asrc + adr
                    sc = jnp.maximum(sc, 0.2 * sc)
                    ee = jnp.exp(sc)
                    hb[e, pl.ds(EECOL, 16)] = ee
                    for k in range(4):
                        hv = hb[e, pl.ds(k * 16, 16)]
                        hb[e, pl.ds(k * 16, 16)] = hv * _vgather(ee, iota // 8 + 2 * k)

            fire_g(0, 0)

            def pair_step(i, _):
                for p in range(2):
                    q = 1 - p
                    j = i * 2 + p
                    drain_g(j, p)

                    @pl.when(j >= 1)
                    def _():
                        drain_s(j - 1, q)

                    @pl.when(j + 1 < NBLKS)
                    def _():
                        fire_g(j + 1, q)

                    compute(p)
                    fire_s(j, p)
                return 0

            lax.fori_loop(0, NBLKS // 2, pair_step, 0)
            drain_s(NBLKS - 1, 1)
            return 0

        lax.fori_loop(0, EW // SBLK, superblock, 0)
        plsc.subcore_barrier()
        pltpu.sync_copy(acc.at[pl.ds(rb, RPT)], pout.at[cc, pl.ds(rb, RPT)])

    zero_acc()
    plsc.subcore_barrier()
    run_hop(tab1, p1out, False)
    plsc.subcore_barrier()
    zero_acc()
    plsc.subcore_barrier()
    run_hop(tab2, p2out, True)


def _sc_edge_passes(srcp, dstp, tab1, tab2, zh):
    mesh = plsc.VectorSubcoreMesh(core_axis_name="c", subcore_axis_name="s")
    f32, i32 = jnp.float32, jnp.int32
    kern = pl.kernel(
        _sc_body,
        out_type=[
            jax.ShapeDtypeStruct((NC, NP, OW), f32),
            jax.ShapeDtypeStruct((NC, NP, OW), f32),
        ],
        mesh=mesh,
        compiler_params=pltpu.CompilerParams(needs_layout_passes=False),
        scratch_types=[
            pltpu.VMEM_SHARED((NP, OW), f32),
            pltpu.VMEM((TCAP,), i32),
            pltpu.VMEM((TCHUNK,), i32),
            pltpu.VMEM((TCHUNK,), i32),
            pltpu.VMEM((SBLK // BLK, BLK), i32),
            pltpu.VMEM((SBLK // BLK, BLK), i32),
            pltpu.VMEM((SBLK // BLK, BLK), i32),
            pltpu.VMEM((BLK, 128), f32),
            pltpu.VMEM((BLK, 128), f32),
            pltpu.VMEM((BLK, 128), f32),
            pltpu.VMEM((BLK, 128), f32),
            pltpu.SemaphoreType.DMA,
            pltpu.SemaphoreType.DMA,
            pltpu.SemaphoreType.DMA,
            pltpu.SemaphoreType.DMA,
        ],
    )
    return kern(srcp, dstp, tab1, tab2, zh)


def _tc_e_body(x_ref, t1_ref, t2_ref, p1a_ref, p1b_ref, p2a_ref, p2b_ref,
               rep_ref, b1_ref, b2_ref, g_ref, bt_ref, o_ref):
    rep = rep_ref[...]

    def hop(t_ref, pa_ref, pb_ref, b_ref):
        t = t_ref[...]
        pa = pa_ref[...]
        pb = pb_ref[...]
        a = t[:, 64:72] + t[:, 72:80]
        w8 = jnp.exp(jnp.maximum(a, 0.2 * a))
        den8 = pa[:, EECOL:EECOL + 8] + pb[:, EECOL:EECOL + 8] + w8
        den64 = jnp.dot(den8, rep, preferred_element_type=jnp.float32)
        w64 = jnp.dot(w8, rep, preferred_element_type=jnp.float32)
        num = pa[:, :64] + pb[:, :64] + w64 * t[:, :64]
        return num / (den64 + 1e-16) + b_ref[...]

    y1 = hop(t1_ref, p1a_ref, p1b_ref, b1_ref)
    y2 = hop(t2_ref, p2a_ref, p2b_ref, b2_ref)
    r = jnp.concatenate([y1, y2], axis=-1) + x_ref[...]
    mu = jnp.mean(r, axis=-1, keepdims=True)
    var = jnp.mean((r - mu) ** 2, axis=-1, keepdims=True)
    o_ref[...] = (r - mu) * lax.rsqrt(var + 1e-5) * g_ref[...] + bt_ref[...]


def _tc_e(x, t1, t2, p1a, p1b, p2a, p2b, rep, b1, b2, g, bt):
    R = _ROWS_TC
    rs = lambda w: pl.BlockSpec((R, w), lambda i: (i, 0))
    full = lambda shape: pl.BlockSpec(shape, lambda i: (0, 0))
    return pl.pallas_call(
        _tc_e_body,
        grid=(N // R,),
        in_specs=[rs(128), rs(128), rs(128), rs(OW), rs(OW), rs(OW), rs(OW),
                  full((8, 64)), full((1, 64)), full((1, 64)),
                  full((1, 128)), full((1, 128))],
        out_specs=rs(128),
        out_shape=jax.ShapeDtypeStruct((N, D_IN), jnp.float32),
    )(x, t1, t2, p1a, p1b, p2a, p2b, rep, b1, b2, g, bt)


def kernel(x, edge_index, W1, att_src1, att_dst1, b1,
           W2, att_src2, att_dst2, b2, ln_gamma, ln_beta):
    f32, i32 = jnp.float32, jnp.int32
    rows = jnp.arange(PER_HOP, dtype=i32)

    def aproj(att_s, att_d):
        ap = jnp.zeros((PER_HOP, PER_HOP), f32)
        ap = ap.at[rows, rows // C].set(att_s.reshape(-1))
        ap = ap.at[rows, 8 + rows // C].set(att_d.reshape(-1))
        return ap

    def adproj(att_d):
        return jnp.zeros((PER_HOP, 16), f32).at[rows, rows // C].set(att_d.reshape(-1))

    AP1 = aproj(att_src1, att_dst1)
    AP2 = aproj(att_src2, att_dst2)
    ADP1, ADP2 = adproj(att_dst1), adproj(att_dst2)

    tab1, adt1, tab2, adt2 = _tc_a(x, W1, AP1, ADP1, W2, AP2, ADP2)

    pad = jnp.zeros((EPAD - E,), i32)
    srcp = jnp.concatenate([edge_index[0], pad])
    dstp = jnp.concatenate([edge_index[1], pad])
    zh = jnp.zeros((RPT, OW), f32)

    p1, p2 = _sc_edge_passes(srcp, dstp, tab1, tab2, zh)

    rep = (rows[None, :] // C == jnp.arange(8, dtype=i32)[:, None]).astype(f32)
    return _tc_e(x, tab1, tab2,
                 p1[0, :N], p1[1, :N], p2[0, :N], p2[1, :N],
                 rep, b1.reshape(1, -1), b2.reshape(1, -1),
                 ln_gamma.reshape(1, -1), ln_beta.reshape(1, -1))


# speedup vs baseline: 1.0020x; 1.0020x over previous
import jax
import jax.numpy as jnp
from jax import lax
from jax.experimental import pallas as pl
from jax.experimental.pallas import tpu as pltpu
from jax.experimental.pallas import tpu_sc as plsc

N = 10000
E = 320000
D_IN = 128
HEADS = 8
C = 8
PER_HOP = 64

NC = 2
NS = 16
NW = NC * NS
BLK = 64
SBLK = 512
EW = 10240
EPAD = EW * NW
NP = 10112
RPT = NP // NS
TCHUNK = 512
TCAP = 10544

OW = 128
EECOL = 112

_ROWS_TC = 1000


def _tc_a_body(x_ref, w1_ref, ap1_ref, adp1_ref, w2_ref, ap2_ref, adp2_ref,
               t1_ref, d1_ref, t2_ref, d2_ref):
    x = x_ref[...]
    for w_ref, ap_ref, adp_ref, t_ref, d_ref in (
            (w1_ref, ap1_ref, adp1_ref, t1_ref, d1_ref),
            (w2_ref, ap2_ref, adp2_ref, t2_ref, d2_ref)):
        h = jnp.dot(x, w_ref[...], preferred_element_type=jnp.float32)
        ab = jnp.dot(h, ap_ref[...], preferred_element_type=jnp.float32)
        t_ref[...] = jnp.concatenate([h, ab], axis=-1)
        d_ref[...] = jnp.dot(h, adp_ref[...], preferred_element_type=jnp.float32)


def _tc_a(x, W1, AP1, ADP1, W2, AP2, ADP2):
    grid = (N // _ROWS_TC,)
    rs = lambda w: pl.BlockSpec((_ROWS_TC, w), lambda i: (i, 0))
    full = lambda shape: pl.BlockSpec(shape, lambda i: (0, 0))
    return pl.pallas_call(
        _tc_a_body,
        grid=grid,
        in_specs=[rs(D_IN), full((D_IN, PER_HOP)), full((PER_HOP, PER_HOP)),
                  full((PER_HOP, 16)), full((D_IN, PER_HOP)),
                  full((PER_HOP, PER_HOP)), full((PER_HOP, 16))],
        out_specs=[rs(128), rs(16), rs(128), rs(16)],
        out_shape=[
            jax.ShapeDtypeStruct((N, 128), jnp.float32),
            jax.ShapeDtypeStruct((N, 16), jnp.float32),
            jax.ShapeDtypeStruct((N, 128), jnp.float32),
            jax.ShapeDtypeStruct((N, 16), jnp.float32),
        ],
    )(x, W1, AP1, ADP1, W2, AP2, ADP2)


def _vgather(x, idx):
    return lax.gather(
        x, idx[:, None],
        lax.GatherDimensionNumbers(offset_dims=(), collapsed_slice_dims=(0,),
                                   start_index_map=(0,)),
        (1,), mode=lax.GatherScatterMode.PROMISE_IN_BOUNDS)


def _sc_body(srcp, dstp, tab1, tab2, zh,
             p1out, p2out, acc,
             tpack_v, tsrc_v, tdst_v,
             gsrcb_v, gdstb_v, idxb_v,
             hrows_a, hrows_b, adst_a, adst_b,
             semg_a, semg_b, sems_a, sems_b):
    cc = lax.axis_index("c")
    ss = lax.axis_index("s")
    wid = cc * NS + ss
    ebase = wid * EW
    rb = ss * RPT
    iota = lax.iota(jnp.int32, 16)
    zv = iota * 0

    def zero_acc():
        pltpu.sync_copy(zh, acc.at[pl.ds(rb, RPT)])

    def zloop(i, _):
        tpack_v[pl.ds(i * 16, 16)] = zv
        return 0

    lax.fori_loop(0, TCAP // 16, zloop, 0, unroll=4)

    def t_chunk(ci, cnt):
        def live(cnt):
            base = pl.multiple_of(ci * TCHUNK, TCHUNK)
            pltpu.sync_copy(srcp.at[pl.ds(base, TCHUNK)], tsrc_v)
            pltpu.sync_copy(dstp.at[pl.ds(base, TCHUNK)], tdst_v)

            def vstep(k, cnt):
                sv = tsrc_v[pl.ds(k * 16, 16)]
                dv = tdst_v[pl.ds(k * 16, 16)]
                m = sv != dv
                cs = plsc.cumsum(m.astype(jnp.int32))
                pos = jnp.where(m, cnt + cs - 1, (TCAP - 16) + iota)
                plsc.store_scatter(tpack_v, [pos], sv + (dv << 16))
                return cnt + cs[15]

            return lax.fori_loop(0, TCHUNK // 16, vstep, cnt)

        return lax.cond(cnt < N, live, lambda cnt: cnt, cnt)

    lax.fori_loop(0, EPAD // TCHUNK, t_chunk, 0)

    hrows = (hrows_a, hrows_b)
    adst = (adst_a, adst_b)
    semg = (semg_a, semg_b)
    sems = (sems_a, sems_b)
    NBLKS = SBLK // BLK

    def run_hop(tab, pout, is_hop2):
        def superblock(sb, _):
            base = pl.multiple_of(ebase + sb * SBLK, SBLK)
            pltpu.sync_copy(srcp.at[pl.ds(base, SBLK)], tsrc_v)
            pltpu.sync_copy(dstp.at[pl.ds(base, SBLK)], tdst_v)

            def prep(k, _):
                sv = tsrc_v[pl.ds(k * 16, 16)]
                dv = tdst_v[pl.ds(k * 16, 16)]
                keep = sv != dv
                if is_hop2:
                    tp = plsc.load_gather(tpack_v, [dv])
                    s2 = tp & 0xFFFF
                    d2 = lax.shift_right_logical(tp, 16)
                    keep = jnp.logical_and(keep, s2 != d2)
                    gs, gd = s2, d2
                else:
                    gs, gd = sv, dv
                j = k // (BLK // 16)
                o = (k % (BLK // 16)) * 16
                gsrcb_v[j, pl.ds(o, 16)] = gs
                gdstb_v[j, pl.ds(o, 16)] = gd
                idxb_v[j, pl.ds(o, 16)] = jnp.where(keep, gd, N)
                return 0

            lax.fori_loop(0, SBLK // 16, prep, 0)

            def fire_g(j, p):
                pltpu.async_copy(tab.at[gsrcb_v.at[j]], hrows[p], semg[p])
                pltpu.async_copy(tab.at[gdstb_v.at[j]], adst[p], semg[p])

            def drain_g(j, p):
                pltpu.make_async_copy(tab.at[gsrcb_v.at[j]], hrows[p], semg[p]).wait()
                pltpu.make_async_copy(tab.at[gdstb_v.at[j]], adst[p], semg[p]).wait()

            def fire_s(j, p):
                pltpu.async_copy(hrows[p], acc.at[idxb_v.at[j]], sems[p], add=True)

            def drain_s(j, p):
                pltpu.make_async_copy(hrows[p], acc.at[idxb_v.at[j]], sems[p]).wait()

            def compute(p):
                hb, ab = hrows[p], adst[p]

                @plsc.parallel_loop(0, BLK, unroll=8)
                def _(e):
                    asrc = hb[e, pl.ds(64, 16)]
                    adv = ab[e, pl.ds(64, 16)]
                    adr = _vgather(adv, (iota + 8) & 15)
                    sc = asrc + adr
                    sc = jnp.maximum(sc, 0.2 * sc)
                    ee = jnp.exp(sc)
                    hb[e, pl.ds(EECOL, 16)] = ee
                    for k in range(4):
                        hv = hb[e, pl.ds(k * 16, 16)]
                        hb[e, pl.ds(k * 16, 16)] = hv * _vgather(ee, iota // 8 + 2 * k)

            fire_g(0, 0)

            def pair_step(i, _):
                for p in range(2):
                    q = 1 - p
                    j = i * 2 + p
                    drain_g(j, p)

                    @pl.when(j >= 1)
                    def _():
                        drain_s(j - 1, q)

                    @pl.when(j + 1 < NBLKS)
                    def _():
                        fire_g(j + 1, q)

                    compute(p)
                    fire_s(j, p)
                return 0

            lax.fori_loop(0, NBLKS // 2, pair_step, 0)
            drain_s(NBLKS - 1, 1)
            return 0

        lax.fori_loop(0, EW // SBLK, superblock, 0)
        plsc.subcore_barrier()
        pltpu.sync_copy(acc.at[pl.ds(rb, RPT)], pout.at[cc, pl.ds(rb, RPT)])

    zero_acc()
    plsc.subcore_barrier()
    run_hop(tab1, p1out, False)
    zero_acc()
    plsc.subcore_barrier()
    run_hop(tab2, p2out, True)


def _sc_edge_passes(srcp, dstp, tab1, tab2, zh):
    mesh = plsc.VectorSubcoreMesh(core_axis_name="c", subcore_axis_name="s")
    f32, i32 = jnp.float32, jnp.int32
    kern = pl.kernel(
        _sc_body,
        out_type=[
            jax.ShapeDtypeStruct((NC, NP, OW), f32),
            jax.ShapeDtypeStruct((NC, NP, OW), f32),
        ],
        mesh=mesh,
        compiler_params=pltpu.CompilerParams(needs_layout_passes=False),
        scratch_types=[
            pltpu.VMEM_SHARED((NP, OW), f32),
            pltpu.VMEM((TCAP,), i32),
            pltpu.VMEM((TCHUNK,), i32),
            pltpu.VMEM((TCHUNK,), i32),
            pltpu.VMEM((SBLK // BLK, BLK), i32),
            pltpu.VMEM((SBLK // BLK, BLK), i32),
            pltpu.VMEM((SBLK // BLK, BLK), i32),
            pltpu.VMEM((BLK, 128), f32),
            pltpu.VMEM((BLK, 128), f32),
            pltpu.VMEM((BLK, 128), f32),
            pltpu.VMEM((BLK, 128), f32),
            pltpu.SemaphoreType.DMA,
            pltpu.SemaphoreType.DMA,
            pltpu.SemaphoreType.DMA,
            pltpu.SemaphoreType.DMA,
        ],
    )
    return kern(srcp, dstp, tab1, tab2, zh)


def _tc_e_body(x_ref, t1_ref, t2_ref, p1a_ref, p1b_ref, p2a_ref, p2b_ref,
               rep_ref, b1_ref, b2_ref, g_ref, bt_ref, o_ref):
    rep = rep_ref[...]

    def hop(t_ref, pa_ref, pb_ref, b_ref):
        t = t_ref[...]
        pa = pa_ref[...]
        pb = pb_ref[...]
        a = t[:, 64:72] + t[:, 72:80]
        w8 = jnp.exp(jnp.maximum(a, 0.2 * a))
        den8 = pa[:, EECOL:EECOL + 8] + pb[:, EECOL:EECOL + 8] + w8
        den64 = jnp.dot(den8, rep, preferred_element_type=jnp.float32)
        w64 = jnp.dot(w8, rep, preferred_element_type=jnp.float32)
        num = pa[:, :64] + pb[:, :64] + w64 * t[:, :64]
        return num / (den64 + 1e-16) + b_ref[...]

    y1 = hop(t1_ref, p1a_ref, p1b_ref, b1_ref)
    y2 = hop(t2_ref, p2a_ref, p2b_ref, b2_ref)
    r = jnp.concatenate([y1, y2], axis=-1) + x_ref[...]
    mu = jnp.mean(r, axis=-1, keepdims=True)
    var = jnp.mean((r - mu) ** 2, axis=-1, keepdims=True)
    o_ref[...] = (r - mu) * lax.rsqrt(var + 1e-5) * g_ref[...] + bt_ref[...]


def _tc_e(x, t1, t2, p1a, p1b, p2a, p2b, rep, b1, b2, g, bt):
    R = _ROWS_TC
    rs = lambda w: pl.BlockSpec((R, w), lambda i: (i, 0))
    full = lambda shape: pl.BlockSpec(shape, lambda i: (0, 0))
    return pl.pallas_call(
        _tc_e_body,
        grid=(N // R,),
        in_specs=[rs(128), rs(128), rs(128), rs(OW), rs(OW), rs(OW), rs(OW),
                  full((8, 64)), full((1, 64)), full((1, 64)),
                  full((1, 128)), full((1, 128))],
        out_specs=rs(128),
        out_shape=jax.ShapeDtypeStruct((N, D_IN), jnp.float32),
    )(x, t1, t2, p1a, p1b, p2a, p2b, rep, b1, b2, g, bt)


def kernel(x, edge_index, W1, att_src1, att_dst1, b1,
           W2, att_src2, att_dst2, b2, ln_gamma, ln_beta):
    f32, i32 = jnp.float32, jnp.int32
    rows = jnp.arange(PER_HOP, dtype=i32)

    def aproj(att_s, att_d):
        ap = jnp.zeros((PER_HOP, PER_HOP), f32)
        ap = ap.at[rows, rows // C].set(att_s.reshape(-1))
        ap = ap.at[rows, 8 + rows // C].set(att_d.reshape(-1))
        return ap

    def adproj(att_d):
        return jnp.zeros((PER_HOP, 16), f32).at[rows, rows // C].set(att_d.reshape(-1))

    AP1 = aproj(att_src1, att_dst1)
    AP2 = aproj(att_src2, att_dst2)
    ADP1, ADP2 = adproj(att_dst1), adproj(att_dst2)

    tab1, adt1, tab2, adt2 = _tc_a(x, W1, AP1, ADP1, W2, AP2, ADP2)

    pad = jnp.zeros((EPAD - E,), i32)
    srcp = jnp.concatenate([edge_index[0], pad])
    dstp = jnp.concatenate([edge_index[1], pad])
    zh = jnp.zeros((RPT, OW), f32)

    p1, p2 = _sc_edge_passes(srcp, dstp, tab1, tab2, zh)

    rep = (rows[None, :] // C == jnp.arange(8, dtype=i32)[:, None]).astype(f32)
    return _tc_e(x, tab1, tab2,
                 p1[0, :N], p1[1, :N], p2[0, :N], p2[1, :N],
                 rep, b1.reshape(1, -1), b2.reshape(1, -1),
                 ln_gamma.reshape(1, -1), ln_beta.reshape(1, -1))


# speedup vs baseline: 1.0332x; 1.0311x over previous
import jax
import jax.numpy as jnp
from jax import lax
from jax.experimental import pallas as pl
from jax.experimental.pallas import tpu as pltpu
from jax.experimental.pallas import tpu_sc as plsc

N = 10000
E = 320000
D_IN = 128
HEADS = 8
C = 8
PER_HOP = 64

NC = 2
NS = 16
NW = NC * NS
BLK = 64
SBLK = 512
EW = 10240
EPAD = EW * NW
NP = 10112
RPT = NP // NS
TCHUNK = 512
TCAP = 10544

OW = 128
EECOL = 112

_ROWS_TC = 1000


def _tc_a_body(x_ref, w1_ref, ap1_ref, w2_ref, ap2_ref, t1_ref, t2_ref):
    x = x_ref[...]
    for w_ref, ap_ref, t_ref in ((w1_ref, ap1_ref, t1_ref),
                                 (w2_ref, ap2_ref, t2_ref)):
        h = jnp.dot(x, w_ref[...], preferred_element_type=jnp.float32)
        ab = jnp.dot(h, ap_ref[...], preferred_element_type=jnp.float32)
        t_ref[...] = jnp.concatenate([h, ab], axis=-1)


def _tc_a(x, W1, AP1, W2, AP2):
    grid = (N // _ROWS_TC,)
    rs = lambda w: pl.BlockSpec((_ROWS_TC, w), lambda i: (i, 0))
    full = lambda shape: pl.BlockSpec(shape, lambda i: (0, 0))
    return pl.pallas_call(
        _tc_a_body,
        grid=grid,
        in_specs=[rs(D_IN), full((D_IN, PER_HOP)), full((PER_HOP, PER_HOP)),
                  full((D_IN, PER_HOP)), full((PER_HOP, PER_HOP))],
        out_specs=[rs(128), rs(128)],
        out_shape=[
            jax.ShapeDtypeStruct((N, 128), jnp.float32),
            jax.ShapeDtypeStruct((N, 128), jnp.float32),
        ],
    )(x, W1, AP1, W2, AP2)


def _vgather(x, idx):
    return lax.gather(
        x, idx[:, None],
        lax.GatherDimensionNumbers(offset_dims=(), collapsed_slice_dims=(0,),
                                   start_index_map=(0,)),
        (1,), mode=lax.GatherScatterMode.PROMISE_IN_BOUNDS)


def _sc_body(srcp, dstp, tab1, tab2, zh,
             p1out, p2out, acc,
             tpack_v, tsrc_v, tdst_v,
             gsrcb_v, gdstb_v, idxb_v,
             hrows_a, hrows_b, adst_a, adst_b,
             semg_a, semg_b, sems_a, sems_b):
    cc = lax.axis_index("c")
    ss = lax.axis_index("s")
    wid = cc * NS + ss
    ebase = wid * EW
    rb = ss * RPT
    iota = lax.iota(jnp.int32, 16)
    zv = iota * 0

    def zero_acc():
        pltpu.sync_copy(zh, acc.at[pl.ds(rb, RPT)])

    def zloop(i, _):
        tpack_v[pl.ds(i * 16, 16)] = zv
        return 0

    lax.fori_loop(0, TCAP // 16, zloop, 0, unroll=4)

    def t_chunk(ci, cnt):
        def live(cnt):
            base = pl.multiple_of(ci * TCHUNK, TCHUNK)
            pltpu.sync_copy(srcp.at[pl.ds(base, TCHUNK)], tsrc_v)
            pltpu.sync_copy(dstp.at[pl.ds(base, TCHUNK)], tdst_v)

            def vstep(k, cnt):
                sv = tsrc_v[pl.ds(k * 16, 16)]
                dv = tdst_v[pl.ds(k * 16, 16)]
                m = sv != dv
                cs = plsc.cumsum(m.astype(jnp.int32))
                pos = jnp.where(m, cnt + cs - 1, (TCAP - 16) + iota)
                plsc.store_scatter(tpack_v, [pos], sv + (dv << 16))
                return cnt + cs[15]

            return lax.fori_loop(0, TCHUNK // 16, vstep, cnt)

        return lax.cond(cnt < N, live, lambda cnt: cnt, cnt)

    lax.fori_loop(0, EPAD // TCHUNK, t_chunk, 0)

    hrows = (hrows_a, hrows_b)
    adst = (adst_a, adst_b)
    semg = (semg_a, semg_b)
    sems = (sems_a, sems_b)
    NBLKS = SBLK // BLK

    def run_hop(tab, pout, is_hop2):
        def superblock(sb, _):
            base = pl.multiple_of(ebase + sb * SBLK, SBLK)
            pltpu.sync_copy(srcp.at[pl.ds(base, SBLK)], tsrc_v)
            pltpu.sync_copy(dstp.at[pl.ds(base, SBLK)], tdst_v)

            def prep(k, _):
                sv = tsrc_v[pl.ds(k * 16, 16)]
                dv = tdst_v[pl.ds(k * 16, 16)]
                keep = sv != dv
                if is_hop2:
                    tp = plsc.load_gather(tpack_v, [dv])
                    s2 = tp & 0xFFFF
                    d2 = lax.shift_right_logical(tp, 16)
                    keep = jnp.logical_and(keep, s2 != d2)
                    gs, gd = s2, d2
                else:
                    gs, gd = sv, dv
                j = k // (BLK // 16)
                o = (k % (BLK // 16)) * 16
                gsrcb_v[j, pl.ds(o, 16)] = gs
                gdstb_v[j, pl.ds(o, 16)] = gd
                idxb_v[j, pl.ds(o, 16)] = jnp.where(keep, gd, N)
                return 0

            lax.fori_loop(0, SBLK // 16, prep, 0)

            def fire_g(j, p):
                pltpu.async_copy(tab.at[gsrcb_v.at[j]], hrows[p], semg[p])
                pltpu.async_copy(tab.at[gdstb_v.at[j]], adst[p], semg[p])

            def drain_g(j, p):
                pltpu.make_async_copy(tab.at[gsrcb_v.at[j]], hrows[p], semg[p]).wait()
                pltpu.make_async_copy(tab.at[gdstb_v.at[j]], adst[p], semg[p]).wait()

            def fire_s(j, p):
                pltpu.async_copy(hrows[p], acc.at[idxb_v.at[j]], sems[p], add=True)

            def drain_s(j, p):
                pltpu.make_async_copy(hrows[p], acc.at[idxb_v.at[j]], sems[p]).wait()

            def compute(p):
                hb, ab = hrows[p], adst[p]

                @plsc.parallel_loop(0, BLK, unroll=8)
                def _(e):
                    asrc = hb[e, pl.ds(64, 16)]
                    adv = ab[e, pl.ds(64, 16)]
                    adr = _vgather(adv, (iota + 8) & 15)
                    sc = asrc + adr
                    sc = jnp.maximum(sc, 0.2 * sc)
                    ee = jnp.exp(sc)
                    hb[e, pl.ds(EECOL, 16)] = ee
                    for k in range(4):
                        hv = hb[e, pl.ds(k * 16, 16)]
                        hb[e, pl.ds(k * 16, 16)] = hv * _vgather(ee, iota // 8 + 2 * k)

            fire_g(0, 0)

            def pair_step(i, _):
                for p in range(2):
                    q = 1 - p
                    j = i * 2 + p
                    drain_g(j, p)

                    @pl.when(j >= 1)
                    def _():
                        drain_s(j - 1, q)

                    @pl.when(j + 1 < NBLKS)
                    def _():
                        fire_g(j + 1, q)

                    compute(p)
                    fire_s(j, p)
                return 0

            lax.fori_loop(0, NBLKS // 2, pair_step, 0)
            drain_s(NBLKS - 1, 1)
            return 0

        lax.fori_loop(0, EW // SBLK, superblock, 0)
        plsc.subcore_barrier()
        pltpu.sync_copy(acc.at[pl.ds(rb, RPT)], pout.at[cc, pl.ds(rb, RPT)])

    zero_acc()
    plsc.subcore_barrier()
    run_hop(tab1, p1out, False)
    zero_acc()
    plsc.subcore_barrier()
    run_hop(tab2, p2out, True)


def _sc_edge_passes(srcp, dstp, tab1, tab2, zh):
    mesh = plsc.VectorSubcoreMesh(core_axis_name="c", subcore_axis_name="s")
    f32, i32 = jnp.float32, jnp.int32
    kern = pl.kernel(
        _sc_body,
        out_type=[
            jax.ShapeDtypeStruct((NC, NP, OW), f32),
            jax.ShapeDtypeStruct((NC, NP, OW), f32),
        ],
        mesh=mesh,
        compiler_params=pltpu.CompilerParams(needs_layout_passes=False),
        scratch_types=[
            pltpu.VMEM_SHARED((NP, OW), f32),
            pltpu.VMEM((TCAP,), i32),
            pltpu.VMEM((TCHUNK,), i32),
            pltpu.VMEM((TCHUNK,), i32),
            pltpu.VMEM((SBLK // BLK, BLK), i32),
            pltpu.VMEM((SBLK // BLK, BLK), i32),
            pltpu.VMEM((SBLK // BLK, BLK), i32),
            pltpu.VMEM((BLK, 128), f32),
            pltpu.VMEM((BLK, 128), f32),
            pltpu.VMEM((BLK, 128), f32),
            pltpu.VMEM((BLK, 128), f32),
            pltpu.SemaphoreType.DMA,
            pltpu.SemaphoreType.DMA,
            pltpu.SemaphoreType.DMA,
            pltpu.SemaphoreType.DMA,
        ],
    )
    return kern(srcp, dstp, tab1, tab2, zh)


def _tc_e_body(x_ref, t1_ref, t2_ref, p1a_ref, p1b_ref, p2a_ref, p2b_ref,
               rep_ref, b1_ref, b2_ref, g_ref, bt_ref, o_ref):
    rep = rep_ref[...]

    def hop(t_ref, pa_ref, pb_ref, b_ref):
        t = t_ref[...]
        pa = pa_ref[...]
        pb = pb_ref[...]
        a = t[:, 64:72] + t[:, 72:80]
        w8 = jnp.exp(jnp.maximum(a, 0.2 * a))
        den8 = pa[:, EECOL:EECOL + 8] + pb[:, EECOL:EECOL + 8] + w8
        den64 = jnp.dot(den8, rep, preferred_element_type=jnp.float32)
        w64 = jnp.dot(w8, rep, preferred_element_type=jnp.float32)
        num = pa[:, :64] + pb[:, :64] + w64 * t[:, :64]
        return num / (den64 + 1e-16) + b_ref[...]

    y1 = hop(t1_ref, p1a_ref, p1b_ref, b1_ref)
    y2 = hop(t2_ref, p2a_ref, p2b_ref, b2_ref)
    r = jnp.concatenate([y1, y2], axis=-1) + x_ref[...]
    mu = jnp.mean(r, axis=-1, keepdims=True)
    var = jnp.mean((r - mu) ** 2, axis=-1, keepdims=True)
    o_ref[...] = (r - mu) * lax.rsqrt(var + 1e-5) * g_ref[...] + bt_ref[...]


def _tc_e(x, t1, t2, p1a, p1b, p2a, p2b, rep, b1, b2, g, bt):
    R = _ROWS_TC
    rs = lambda w: pl.BlockSpec((R, w), lambda i: (i, 0))
    full = lambda shape: pl.BlockSpec(shape, lambda i: (0, 0))
    return pl.pallas_call(
        _tc_e_body,
        grid=(N // R,),
        in_specs=[rs(128), rs(128), rs(128), rs(OW), rs(OW), rs(OW), rs(OW),
                  full((8, 64)), full((1, 64)), full((1, 64)),
                  full((1, 128)), full((1, 128))],
        out_specs=rs(128),
        out_shape=jax.ShapeDtypeStruct((N, D_IN), jnp.float32),
    )(x, t1, t2, p1a, p1b, p2a, p2b, rep, b1, b2, g, bt)


def kernel(x, edge_index, W1, att_src1, att_dst1, b1,
           W2, att_src2, att_dst2, b2, ln_gamma, ln_beta):
    f32, i32 = jnp.float32, jnp.int32
    rows = jnp.arange(PER_HOP, dtype=i32)

    def aproj(att_s, att_d):
        ap = jnp.zeros((PER_HOP, PER_HOP), f32)
        ap = ap.at[rows, rows // C].set(att_s.reshape(-1))
        ap = ap.at[rows, 8 + rows // C].set(att_d.reshape(-1))
        return ap

    AP1 = aproj(att_src1, att_dst1)
    AP2 = aproj(att_src2, att_dst2)

    tab1, tab2 = _tc_a(x, W1, AP1, W2, AP2)

    pad = jnp.zeros((EPAD - E,), i32)
    srcp = jnp.concatenate([edge_index[0], pad])
    dstp = jnp.concatenate([edge_index[1], pad])
    zh = jnp.zeros((RPT, OW), f32)

    p1, p2 = _sc_edge_passes(srcp, dstp, tab1, tab2, zh)

    rep = (rows[None, :] // C == jnp.arange(8, dtype=i32)[:, None]).astype(f32)
    return _tc_e(x, tab1, tab2,
                 p1[0, :N], p1[1, :N], p2[0, :N], p2[1, :N],
                 rep, b1.reshape(1, -1), b2.reshape(1, -1),
                 ln_gamma.reshape(1, -1), ln_beta.reshape(1, -1))
